# Initial kernel scaffold; baseline (speedup 1.0000x reference)
import jax
import jax.numpy as jnp
from jax.experimental import pallas as pl


def kernel(hidden_states, Wr, W1, W2):
    raise NotImplementedError("write your pallas kernel here")


# speedup vs baseline: 2.6210x; 2.6210x over previous
import dataclasses
import functools

import jax
import jax.numpy as jnp
from jax import lax
from jax.experimental import pallas as pl
from jax.experimental.pallas import tpu as pltpu
from jax.experimental.pallas import tpu_sc as plsc

E = 8
D = 1024
F = 4096
T = 4096

BT_R = 512
BT_C = 256
FC = 2048
NF = F // FC
P = T + E * BT_C
NB = P // BT_C

NW = 32
CH = 32
BPW = T // NW


def _router_body(x_ref, wr_ref, logits_ref, pmax_ref, eidx_ref, hist_ref):
    b = pl.program_id(0)
    logits = jnp.dot(x_ref[...], wr_ref[...], preferred_element_type=jnp.float32)
    logits_ref[...] = logits
    m = jnp.max(logits, axis=-1, keepdims=True)
    u = jnp.exp(logits - m)
    s = jnp.sum(u, axis=-1, keepdims=True)
    p = u / s
    pmax_ref[...] = jnp.max(p, axis=-1, keepdims=True)
    idx = jnp.argmax(p, axis=-1)[:, None].astype(jnp.int32)
    eidx_ref[...] = idx
    onehot = (idx == lax.broadcasted_iota(jnp.int32, (1, E), 1)).astype(jnp.float32)
    hsum = jnp.sum(onehot, axis=0, keepdims=True)

    @pl.when(b == 0)
    def _():
        hist_ref[...] = hsum

    @pl.when(b != 0)
    def _():
        hist_ref[...] += hsum


def _positions_body(eidx_ref, hist_ref, pos_ref, offs_ref, acc_ref):
    b = pl.program_id(0)

    @pl.when(b == 0)
    def _():
        cnt = hist_ref[...]
        padded = jnp.ceil(cnt / BT_C) * BT_C
        row = lax.broadcasted_iota(jnp.int32, (E, E), 0)
        col = lax.broadcasted_iota(jnp.int32, (E, E), 1)
        strict = (row < col).astype(jnp.float32)
        offs_ref[...] = jnp.sum(padded.reshape(E, 1) * strict, axis=0, keepdims=True)
        acc_ref[...] = jnp.zeros((1, E), jnp.float32)

    idx = eidx_ref[...]
    onehot = (idx == lax.broadcasted_iota(jnp.int32, (1, E), 1)).astype(jnp.float32)
    r = lax.broadcasted_iota(jnp.int32, (BT_R, BT_R), 0)
    c = lax.broadcasted_iota(jnp.int32, (BT_R, BT_R), 1)
    tril = (r >= c).astype(jnp.float32)
    cums = jax.lax.dot(tril, onehot, precision=jax.lax.Precision.HIGHEST)
    base = offs_ref[...] + acc_ref[...]
    posf = jnp.sum(onehot * (base + cums), axis=-1, keepdims=True) - 1.0
    pos_ref[...] = posf.astype(jnp.int32)
    acc_ref[...] += jnp.sum(onehot, axis=0, keepdims=True)


def _ffn_body(meta_ref, xs_ref, w1_ref, w2_ref, ps_ref, ys_ref):
    f = pl.program_id(1)
    b = pl.program_id(0)
    used = meta_ref[NB]

    @pl.when(b < used)
    def _():
        h = jnp.maximum(
            jnp.dot(xs_ref[...], w1_ref[0], preferred_element_type=jnp.float32), 0.0)
        part = jnp.dot(h, w2_ref[0], preferred_element_type=jnp.float32)

        @pl.when(f == 0)
        def _():
            ys_ref[...] = part

        @pl.when(f > 0)
        def _():
            ys_ref[...] += part

        @pl.when(f == NF - 1)
        def _():
            ys_ref[...] *= ps_ref[...]


def _dispatch_body(x_hbm, pos_hbm, pmax_hbm, xs_hbm, ps_hbm,
                   rows_v, idx_v, posf_v, pm_v, ps_v):
    wid = lax.axis_index("s") * 2 + lax.axis_index("c")
    base = wid * BPW

    @pl.loop(0, BPW // CH)
    def _(j):
        off = base + j * CH
        pltpu.sync_copy(pos_hbm.at[pl.ds(off, CH)], idx_v)
        pltpu.sync_copy(x_hbm.at[pl.ds(off, CH)], rows_v)
        pltpu.sync_copy(rows_v, xs_hbm.at[idx_v])

    @pl.when(wid == 0)
    def _():
        pltpu.sync_copy(pos_hbm, posf_v)
        pltpu.sync_copy(pmax_hbm, pm_v)

        @pl.loop(0, T // 16)
        def _(k):
            i16 = posf_v[pl.ds(k * 16, 16)]
            v16 = pm_v[pl.ds(k * 16, 16)]
            plsc.store_scatter(ps_v, [i16], v16)

        pltpu.sync_copy(ps_v, ps_hbm)


def _collect_body(ys_hbm, pos_hbm, out_hbm, rows_v, idx_v):
    wid = lax.axis_index("s") * 2 + lax.axis_index("c")
    base = wid * BPW

    @pl.loop(0, BPW // CH)
    def _(j):
        off = base + j * CH
        pltpu.sync_copy(pos_hbm.at[pl.ds(off, CH)], idx_v)
        pltpu.sync_copy(ys_hbm.at[idx_v], rows_v)
        pltpu.sync_copy(rows_v, out_hbm.at[pl.ds(off, CH)])


@functools.cache
def _sc_kernels():
    mesh = plsc.VectorSubcoreMesh(core_axis_name="c", subcore_axis_name="s")
    cp = pltpu.CompilerParams()
    if "needs_layout_passes" in pltpu.CompilerParams.__dataclass_fields__:
        cp = dataclasses.replace(cp, needs_layout_passes=False)
    dispatch = pl.kernel(
        _dispatch_body,
        mesh=mesh,
        compiler_params=cp,
        out_type=(
            jax.ShapeDtypeStruct((P, D), jnp.float32),
            jax.ShapeDtypeStruct((P,), jnp.float32),
        ),
        scratch_types=[
            pltpu.VMEM((CH, D), jnp.float32),
            pltpu.VMEM((CH,), jnp.int32),
            pltpu.VMEM((T,), jnp.int32),
            pltpu.VMEM((T,), jnp.float32),
            pltpu.VMEM((P,), jnp.float32),
        ],
    )
    collect = pl.kernel(
        _collect_body,
        mesh=mesh,
        out_type=jax.ShapeDtypeStruct((T, D), jnp.float32),
        scratch_types=[
            pltpu.VMEM((CH, D), jnp.float32),
            pltpu.VMEM((CH,), jnp.int32),
        ],
    )
    return dispatch, collect


def _router_call(x, Wr):
    return pl.pallas_call(
        _router_body,
        grid=(T // BT_R,),
        in_specs=[
            pl.BlockSpec((BT_R, D), lambda b: (b, 0)),
            pl.BlockSpec((D, E), lambda b: (0, 0)),
        ],
        out_specs=[
            pl.BlockSpec((BT_R, E), lambda b: (b, 0)),
            pl.BlockSpec((BT_R, 1), lambda b: (b, 0)),
            pl.BlockSpec((BT_R, 1), lambda b: (b, 0)),
            pl.BlockSpec((1, E), lambda b: (0, 0)),
        ],
        out_shape=[
            jax.ShapeDtypeStruct((T, E), jnp.float32),
            jax.ShapeDtypeStruct((T, 1), jnp.float32),
            jax.ShapeDtypeStruct((T, 1), jnp.int32),
            jax.ShapeDtypeStruct((1, E), jnp.float32),
        ],
        compiler_params=pltpu.CompilerParams(
            dimension_semantics=("arbitrary",)),
    )(x, Wr)


def _positions_call(eidx, hist):
    return pl.pallas_call(
        _positions_body,
        grid=(T // BT_R,),
        in_specs=[
            pl.BlockSpec((BT_R, 1), lambda b: (b, 0)),
            pl.BlockSpec((1, E), lambda b: (0, 0)),
        ],
        out_specs=pl.BlockSpec((BT_R, 1), lambda b: (b, 0)),
        out_shape=jax.ShapeDtypeStruct((T, 1), jnp.int32),
        scratch_shapes=[
            pltpu.VMEM((1, E), jnp.float32),
            pltpu.VMEM((1, E), jnp.float32),
        ],
        compiler_params=pltpu.CompilerParams(
            dimension_semantics=("arbitrary",)),
    )(eidx, hist)


def _ffn_call(meta, xs, W1, W2, ps):
    grid_spec = pltpu.PrefetchScalarGridSpec(
        num_scalar_prefetch=1,
        grid=(NB, NF),
        in_specs=[
            pl.BlockSpec((BT_C, D), lambda b, f, m: (b, 0)),
            pl.BlockSpec((1, D, FC),
                         lambda b, f, m: (m[b], 0, jnp.where(b < m[NB], f, 0))),
            pl.BlockSpec((1, FC, D),
                         lambda b, f, m: (m[b], jnp.where(b < m[NB], f, 0), 0)),
            pl.BlockSpec((BT_C, 1), lambda b, f, m: (b, 0)),
        ],
        out_specs=pl.BlockSpec((BT_C, D), lambda b, f, m: (b, 0)),
    )
    return pl.pallas_call(
        _ffn_body,
        grid_spec=grid_spec,
        out_shape=jax.ShapeDtypeStruct((P, D), jnp.float32),
        compiler_params=pltpu.CompilerParams(
            dimension_semantics=("arbitrary", "arbitrary")),
    )(meta, xs, W1, W2, ps)


def kernel(hidden_states, Wr, W1, W2):
    Bv, Sv, Dv = hidden_states.shape
    x = hidden_states.reshape(-1, Dv)

    logits, pmax, eidx, hist = _router_call(x, Wr)
    pos = _positions_call(eidx, hist)

    cnt = hist.reshape(E).astype(jnp.int32)
    nblk = (cnt + BT_C - 1) // BT_C
    cum = jnp.cumsum(nblk)
    be = jnp.minimum(
        jnp.sum((jnp.arange(NB)[:, None] >= cum[None, :]).astype(jnp.int32), axis=1),
        E - 1).astype(jnp.int32)
    meta = jnp.concatenate([be, cum[E - 1:]], axis=0)

    dispatch, collect = _sc_kernels()
    xs, ps = dispatch(x, pos.reshape(T), pmax.reshape(T))
    ys = _ffn_call(meta, xs, W1, W2, ps.reshape(P, 1))
    out = collect(ys, pos.reshape(T))

    return (out.reshape(Bv, Sv, Dv),
            (logits.reshape(Bv, Sv, E), eidx.reshape(Bv, Sv)))
